# Initial kernel scaffold; baseline (speedup 1.0000x reference)
import jax
import jax.numpy as jnp
from jax.experimental import pallas as pl


def kernel(query, halting_prob, remainders, n_updates, score_mask, tape_tokens):
    raise NotImplementedError("write your pallas kernel here")


# speedup vs baseline: 12.4634x; 12.4634x over previous
import functools
import math

import jax
import jax.numpy as jnp
from jax.experimental import pallas as pl
from jax.experimental.pallas import tpu as pltpu

_FEATURES = 1024
_SPLIT_TT = 2
_NUM_TAPE = 1024
_NUM_PER_STEP = 8
_THRESHOLD = 2.0
_KEYDIM = _FEATURES // _SPLIT_TT
_K = int(_NUM_TAPE // _THRESHOLD)


def _monotone_key(x):
    u = jax.lax.bitcast_convert_type(x, jnp.uint32)
    neg = (u >> jnp.uint32(31)).astype(jnp.bool_)
    flip = jnp.where(neg, jnp.uint32(0xFFFFFFFF), jnp.uint32(0x80000000))
    return u ^ flip


def _topk_mask(key_u, lane_idx, k):
    b = key_u.shape[0]
    kk = jnp.int32(k)
    v = jnp.zeros((b, 1), jnp.uint32)
    for bit in range(31, -1, -1):
        cand = v | jnp.uint32(1 << bit)
        cnt = jnp.sum((key_u >= cand).astype(jnp.int32), axis=1, keepdims=True)
        v = jnp.where(cnt >= kk, cand, v)
    gt = key_u > v
    n_gt = jnp.sum(gt.astype(jnp.int32), axis=1, keepdims=True)
    need = kk - n_gt
    tie = key_u == v
    jb = jnp.zeros((b, 1), jnp.int32)
    for bit in range(10, -1, -1):
        cand = jb | jnp.int32(1 << bit)
        cnt = jnp.sum((tie & (lane_idx < cand)).astype(jnp.int32), axis=1,
                      keepdims=True)
        jb = jnp.where(cnt < need, cand, jb)
    sel = gt | (tie & (lane_idx <= jb))
    return sel.astype(jnp.float32)


def _act_body(q_ref, hp_ref, rem_ref, nup_ref, mask_ref, tape_ref,
              qout_ref, hpout_ref, remout_ref, nupout_ref, maskout_ref,
              tokout_ref):
    c = 1.0 / math.sqrt(_KEYDIM)
    keys = tape_ref[:, :_KEYDIM]
    scores = jax.lax.dot_general(
        q_ref[...], keys, (((1,), (1,)), ((), ())),
        preferred_element_type=jnp.float32)
    masked = scores - mask_ref[...] * 1e9
    key_u = _monotone_key(masked)
    lane = jax.lax.broadcasted_iota(jnp.int32, masked.shape, 1)
    ind512 = _topk_mask(key_u, lane, _K)
    ind8 = _topk_mask(key_u, lane, _NUM_PER_STEP)

    logits = scores[0:1, :] * c
    lmax = jnp.max(logits)
    e = ind512 * jnp.exp(logits - lmax)
    z = jnp.sum(e, axis=1, keepdims=True)
    w = e / z
    sum_w = jnp.sum(w * ind8, axis=1, keepdims=True)
    entropy = 1.0 - jnp.sum(w * w, axis=1, keepdims=True)

    hp = hp_ref[...]
    still0 = (hp < _THRESHOLD).astype(jnp.float32)
    new_halted = (hp + sum_w >= _THRESHOLD).astype(jnp.float32) * still0
    still = still0 - new_halted
    remout_ref[...] = rem_ref[...] + (new_halted + still) * entropy
    hp1 = hp + sum_w * still
    hpout_ref[...] = hp1 + new_halted * (_THRESHOLD - hp1)
    nupout_ref[...] = nup_ref[...] + still + new_halted

    tok = jax.lax.dot_general(
        w, tape_ref[...], (((1,), (0,)), ((), ())),
        preferred_element_type=jnp.float32)
    tokout_ref[...] = tok
    qout_ref[...] = tok[:, :_KEYDIM]
    maskout_ref[...] = mask_ref[...] + ind512


@jax.jit
def kernel(query, halting_prob, remainders, n_updates, score_mask,
           tape_tokens):
    batch = query.shape[0]
    col = lambda x: x.reshape(batch, 1)
    f32 = jnp.float32
    outs = pl.pallas_call(
        _act_body,
        out_shape=(
            jax.ShapeDtypeStruct((batch, _KEYDIM), f32),
            jax.ShapeDtypeStruct((batch, 1), f32),
            jax.ShapeDtypeStruct((batch, 1), f32),
            jax.ShapeDtypeStruct((batch, 1), f32),
            jax.ShapeDtypeStruct((batch, _NUM_TAPE), f32),
            jax.ShapeDtypeStruct((batch, _NUM_TAPE), f32),
        ),
    )(query, col(halting_prob), col(remainders), col(n_updates), score_mask,
      tape_tokens)
    q_out, hp_out, rem_out, nup_out, mask_out, tok_out = outs
    return (q_out, hp_out.reshape(batch), rem_out.reshape(batch),
            nup_out.reshape(batch), mask_out, tok_out.reshape(batch, 1,
                                                              _NUM_TAPE))


# speedup vs baseline: 12.4930x; 1.0024x over previous
import functools
import math

import jax
import jax.numpy as jnp
from jax.experimental import pallas as pl
from jax.experimental.pallas import tpu as pltpu

_FEATURES = 1024
_SPLIT_TT = 2
_NUM_TAPE = 1024
_NUM_PER_STEP = 8
_THRESHOLD = 2.0
_KEYDIM = _FEATURES // _SPLIT_TT
_K = int(_NUM_TAPE // _THRESHOLD)


def _monotone_key(x):
    u = jax.lax.bitcast_convert_type(x, jnp.uint32)
    neg = (u >> jnp.uint32(31)).astype(jnp.bool_)
    flip = jnp.where(neg, jnp.uint32(0xFFFFFFFF), jnp.uint32(0x80000000))
    return u ^ flip


def _topk_two_masks(key_u, lane_idx, k1, k2):
    b = key_u.shape[0]
    kk1, kk2 = jnp.int32(k1), jnp.int32(k2)
    v1 = jnp.zeros((b, 1), jnp.uint32)
    v2 = jnp.zeros((b, 1), jnp.uint32)
    for bit in range(31, -1, -1):
        c1 = v1 | jnp.uint32(1 << bit)
        c2 = v2 | jnp.uint32(1 << bit)
        cnt1 = jnp.sum((key_u >= c1).astype(jnp.int32), axis=1, keepdims=True)
        cnt2 = jnp.sum((key_u >= c2).astype(jnp.int32), axis=1, keepdims=True)
        v1 = jnp.where(cnt1 >= kk1, c1, v1)
        v2 = jnp.where(cnt2 >= kk2, c2, v2)

    def finish(v, kk):
        gt = key_u > v
        n_gt = jnp.sum(gt.astype(jnp.int32), axis=1, keepdims=True)
        need = kk - n_gt
        tie = key_u == v
        return gt, tie, need

    gt1, tie1, need1 = finish(v1, kk1)
    gt2, tie2, need2 = finish(v2, kk2)
    jb1 = jnp.zeros((b, 1), jnp.int32)
    jb2 = jnp.zeros((b, 1), jnp.int32)
    for bit in range(10, -1, -1):
        c1 = jb1 | jnp.int32(1 << bit)
        c2 = jb2 | jnp.int32(1 << bit)
        cnt1 = jnp.sum((tie1 & (lane_idx < c1)).astype(jnp.int32), axis=1,
                       keepdims=True)
        cnt2 = jnp.sum((tie2 & (lane_idx < c2)).astype(jnp.int32), axis=1,
                       keepdims=True)
        jb1 = jnp.where(cnt1 < need1, c1, jb1)
        jb2 = jnp.where(cnt2 < need2, c2, jb2)
    sel1 = gt1 | (tie1 & (lane_idx <= jb1))
    sel2 = gt2 | (tie2 & (lane_idx <= jb2))
    return sel1.astype(jnp.float32), sel2.astype(jnp.float32)


def _act_body(q_ref, hp_ref, rem_ref, nup_ref, mask_ref, tape_ref,
              qout_ref, hpout_ref, remout_ref, nupout_ref, maskout_ref,
              tokout_ref):
    c = 1.0 / math.sqrt(_KEYDIM)
    keys = tape_ref[:, :_KEYDIM]
    scores = jax.lax.dot_general(
        q_ref[...], keys, (((1,), (1,)), ((), ())),
        preferred_element_type=jnp.float32)
    masked = scores - mask_ref[...] * 1e9
    key_u = _monotone_key(masked)
    lane = jax.lax.broadcasted_iota(jnp.int32, masked.shape, 1)
    ind512, ind8 = _topk_two_masks(key_u, lane, _K, _NUM_PER_STEP)

    logits = scores[0:1, :] * c
    lmax = jnp.max(logits)
    e = ind512 * jnp.exp(logits - lmax)
    z = jnp.sum(e, axis=1, keepdims=True)
    w = e / z
    sum_w = jnp.sum(w * ind8, axis=1, keepdims=True)
    entropy = 1.0 - jnp.sum(w * w, axis=1, keepdims=True)

    hp = hp_ref[...]
    still0 = (hp < _THRESHOLD).astype(jnp.float32)
    new_halted = (hp + sum_w >= _THRESHOLD).astype(jnp.float32) * still0
    still = still0 - new_halted
    remout_ref[...] = rem_ref[...] + (new_halted + still) * entropy
    hp1 = hp + sum_w * still
    hpout_ref[...] = hp1 + new_halted * (_THRESHOLD - hp1)
    nupout_ref[...] = nup_ref[...] + still + new_halted

    tok = jax.lax.dot_general(
        w, tape_ref[...], (((1,), (0,)), ((), ())),
        preferred_element_type=jnp.float32)
    tokout_ref[...] = tok
    qout_ref[...] = tok[:, :_KEYDIM]
    maskout_ref[...] = mask_ref[...] + ind512


@jax.jit
def kernel(query, halting_prob, remainders, n_updates, score_mask,
           tape_tokens):
    batch = query.shape[0]
    col = lambda x: x.reshape(batch, 1)
    f32 = jnp.float32
    outs = pl.pallas_call(
        _act_body,
        out_shape=(
            jax.ShapeDtypeStruct((batch, _KEYDIM), f32),
            jax.ShapeDtypeStruct((batch, 1), f32),
            jax.ShapeDtypeStruct((batch, 1), f32),
            jax.ShapeDtypeStruct((batch, 1), f32),
            jax.ShapeDtypeStruct((batch, _NUM_TAPE), f32),
            jax.ShapeDtypeStruct((batch, _NUM_TAPE), f32),
        ),
    )(query, col(halting_prob), col(remainders), col(n_updates), score_mask,
      tape_tokens)
    q_out, hp_out, rem_out, nup_out, mask_out, tok_out = outs
    return (q_out, hp_out.reshape(batch), rem_out.reshape(batch),
            nup_out.reshape(batch), mask_out, tok_out.reshape(batch, 1,
                                                              _NUM_TAPE))
